# baseline (speedup 1.0000x reference)
import jax
import jax.numpy as jnp
from jax.experimental import pallas as pl

MASKED_ID = 103
LN_EPS = 1e-12
S_BLK = 256


def _body(det_ref, emb_ref, pos_ref, word_ref, type_ref, gam_ref, bet_ref,
          hid_ref, sco_ref):
    row = word_ref[MASKED_ID % 8:MASKED_ID % 8 + 1, :] + type_ref[0:1, :]
    x = pos_ref[...] + row
    mean = jnp.mean(x, axis=1, keepdims=True)
    d = x - mean
    var = jnp.mean(d * d, axis=1, keepdims=True)
    m = d * jax.lax.rsqrt(var + LN_EPS) * gam_ref[...] + bet_ref[...]
    sl = det_ref[...][:, 0, :]
    sco_ref[:, 0:1, :] = (1.0 - sl)[:, None, :]
    sco_ref[:, 1:2, :] = sl[:, None, :]
    s = sl[:, :, None]
    hid_ref[...] = (1.0 - s) * emb_ref[...] + s * m[None]


def kernel(detector_scores, embeddings, word_table, pos_table, type_table,
           ln_gamma, ln_beta):
    B, S, _ = detector_scores.shape
    H = embeddings.shape[-1]
    n = S // S_BLK
    gamma2 = ln_gamma.reshape(1, H)
    beta2 = ln_beta.reshape(1, H)
    det2 = detector_scores.transpose(0, 2, 1)
    wblk = MASKED_ID // 8

    grid_spec = pl.GridSpec(
        grid=(n,),
        in_specs=[
            pl.BlockSpec((B, 1, S_BLK), lambda i: (0, 0, i)),
            pl.BlockSpec((B, S_BLK, H), lambda i: (0, i, 0)),
            pl.BlockSpec((S_BLK, H), lambda i: (i, 0)),
            pl.BlockSpec((8, H), lambda i: (wblk, 0)),
            pl.BlockSpec((2, H), lambda i: (0, 0)),
            pl.BlockSpec((1, H), lambda i: (0, 0)),
            pl.BlockSpec((1, H), lambda i: (0, 0)),
        ],
        out_specs=[
            pl.BlockSpec((B, S_BLK, H), lambda i: (0, i, 0)),
            pl.BlockSpec((B, 2, S_BLK), lambda i: (0, 0, i)),
        ],
    )
    hidden, scores_t = pl.pallas_call(
        _body,
        grid_spec=grid_spec,
        out_shape=[
            jax.ShapeDtypeStruct((B, S, H), jnp.float32),
            jax.ShapeDtypeStruct((B, 2, S), jnp.float32),
        ],
    )(det2, embeddings, pos_table, word_table, type_table, gamma2, beta2)
    return (hidden, scores_t.transpose(0, 2, 1))
